# Initial kernel scaffold; baseline (speedup 1.0000x reference)
import jax
import jax.numpy as jnp
from jax.experimental import pallas as pl


def kernel(hidden_states, w_router, w_gate, w_up, w_down):
    raise NotImplementedError("write your pallas kernel here")


# speedup vs baseline: 1.6450x; 1.6450x over previous
import functools

import jax
import jax.numpy as jnp
from jax import lax
from jax.experimental import pallas as pl
from jax.experimental.pallas import tpu as pltpu
from jax.experimental.pallas import tpu_sc as plsc

E = 8
K = 2
H = 1024
FF = 4096
T = 2048
NP = T * K
BBLK = 256
NBLK = NP // BBLK + E
NB = NBLK * BBLK
FT = 1024
NJ = FF // FT

_LANES = 128


def _router_body(x_ref, wr_ref, logits_ref, slots_ref, wp_ref, be_ref):
    x = x_ref[...]
    l = jnp.dot(x, wr_ref[...], preferred_element_type=jnp.float32)
    logits_ref[...] = l

    lane = lax.broadcasted_iota(jnp.int32, (T, _LANES), 1).astype(jnp.float32)
    valid = lane < float(E)
    lm = jnp.where(valid, l, -1e30)
    m = jnp.max(lm, axis=1, keepdims=True)
    ex = jnp.where(valid, jnp.exp(lm - m), 0.0)
    s = jnp.sum(ex, axis=1, keepdims=True)
    a = ex / s

    v1 = jnp.max(a, axis=1, keepdims=True)
    i1 = jnp.min(jnp.where((a == v1) & valid, lane, 1e9), axis=1, keepdims=True)
    a2 = jnp.where(lane == i1, -1.0, a)
    v2 = jnp.max(a2, axis=1, keepdims=True)
    i2 = jnp.min(jnp.where((a2 == v2) & valid, lane, 1e9), axis=1, keepdims=True)
    wsum = v1 + v2
    w1 = v1 / wsum
    w2 = v2 / wsum

    oh = (jnp.where(lane == i1, 1.0, 0.0) + jnp.where(lane == i2, 1.0, 0.0))

    inc = oh
    sh = 1
    while sh < T:
        z = jnp.zeros((sh, _LANES), jnp.float32)
        inc = inc + jnp.concatenate([z, inc[: T - sh, :]], axis=0)
        sh *= 2
    exc = inc - oh

    g = inc[T - 1 : T, :]
    p = jnp.ceil(g / float(BBLK)) * float(BBLK)
    r_iota = lax.broadcasted_iota(jnp.int32, (_LANES, _LANES), 0).astype(jnp.float32)
    c_iota = lax.broadcasted_iota(jnp.int32, (_LANES, _LANES), 1).astype(jnp.float32)
    upper = jnp.where(r_iota < c_iota, 1.0, 0.0)
    off = jnp.dot(p, upper, preferred_element_type=jnp.float32)
    ends = off + p

    rank1 = jnp.sum(jnp.where(lane == i1, exc, 0.0), axis=1, keepdims=True)
    rank2 = jnp.sum(jnp.where(lane == i2, exc + jnp.where(lane == i1, 1.0, 0.0), 0.0),
                    axis=1, keepdims=True)
    offb = jnp.broadcast_to(off, (T, _LANES))
    off1 = jnp.sum(jnp.where(lane == i1, offb, 0.0), axis=1, keepdims=True)
    off2 = jnp.sum(jnp.where(lane == i2, offb, 0.0), axis=1, keepdims=True)
    slot1 = off1 + rank1
    slot2 = off2 + rank2

    lane0 = lane == 0.0
    lane1 = lane == 1.0
    slots = jnp.where(lane0, slot1, 0.0) + jnp.where(lane1, slot2, 0.0)
    slots_ref[...] = slots.astype(jnp.int32)
    wp = jnp.where(lane0, w1, 0.0) + jnp.where(lane1, w2, 0.0)
    wp_ref[...] = wp

    endsm = jnp.where(c_iota[0:1, :] < float(E), ends, 1e30)
    endsb = jnp.broadcast_to(endsm, (_LANES, _LANES))
    istart = r_iota * float(BBLK)
    cnt = jnp.sum(jnp.where(endsb <= istart, 1.0, 0.0), axis=1, keepdims=True)
    be = jnp.minimum(cnt, float(E - 1)).astype(jnp.int32)
    be_ref[...] = jnp.broadcast_to(be, (_LANES, _LANES))


def _router_call(x, wr_pad):
    return pl.pallas_call(
        _router_body,
        out_shape=(
            jax.ShapeDtypeStruct((T, _LANES), jnp.float32),
            jax.ShapeDtypeStruct((T, _LANES), jnp.int32),
            jax.ShapeDtypeStruct((T, _LANES), jnp.float32),
            jax.ShapeDtypeStruct((_LANES, _LANES), jnp.int32),
        ),
    )(x, wr_pad)


_PPW = NP // 32
_CH = 32


def _make_dispatch():
    mesh = plsc.VectorSubcoreMesh(core_axis_name="c", subcore_axis_name="s")

    @functools.partial(
        pl.kernel,
        mesh=mesh,
        out_type=(
            jax.ShapeDtypeStruct((NB, H), jnp.float32),
            jax.ShapeDtypeStruct((NB, _LANES), jnp.float32),
        ),
        scratch_types=[
            pltpu.VMEM((_CH,), jnp.int32),
            pltpu.VMEM((_CH,), jnp.int32),
            pltpu.VMEM((_CH, H), jnp.float32),
            pltpu.VMEM((_CH, _LANES), jnp.float32),
            pltpu.SemaphoreType.DMA,
        ],
    )
    def disp(x_hbm, src_hbm, slots_hbm, w128_hbm, xs_hbm, ws_hbm,
             src_v, slot_v, rows_v, wrow_v, sem):
        wid = lax.axis_index("s") * 2 + lax.axis_index("c")
        base0 = wid * _PPW

        def step(c, _):
            base = base0 + c * _CH
            pltpu.sync_copy(slots_hbm.at[pl.ds(base, _CH)], slot_v)
            pltpu.sync_copy(src_hbm.at[pl.ds(base, _CH)], src_v)
            pltpu.async_copy(x_hbm.at[src_v], rows_v, sem).wait()
            pltpu.sync_copy(w128_hbm.at[pl.ds(base, _CH)], wrow_v)
            pltpu.async_copy(rows_v, xs_hbm.at[slot_v], sem).wait()
            pltpu.async_copy(wrow_v, ws_hbm.at[slot_v], sem).wait()
            return 0

        lax.fori_loop(0, _PPW // _CH, step, 0)

    return disp


def _gmlp_body(be_ref, xs_ref, wg_ref, wu_ref, wd_ref, ws_ref, out_ref):
    j = pl.program_id(1)
    x = xs_ref[...]
    g = jnp.dot(x, wg_ref[0], preferred_element_type=jnp.float32)
    u = jnp.dot(x, wu_ref[0], preferred_element_type=jnp.float32)
    act = (g * jax.nn.sigmoid(g)) * u
    part = jnp.dot(act, wd_ref[0], preferred_element_type=jnp.float32)

    @pl.when(j == 0)
    def _():
        out_ref[...] = part

    @pl.when(j > 0)
    def _():
        out_ref[...] += part

    @pl.when(j == NJ - 1)
    def _():
        out_ref[...] = out_ref[...] * ws_ref[0][:, 0:1]


def _gmlp_call(be, xs, w_gate, w_up, w_down, ws3):
    grid_spec = pltpu.PrefetchScalarGridSpec(
        num_scalar_prefetch=1,
        grid=(NBLK, NJ),
        in_specs=[
            pl.BlockSpec((BBLK, H), lambda i, j, be: (i, 0)),
            pl.BlockSpec((1, H, FT), lambda i, j, be: (be[i], 0, j)),
            pl.BlockSpec((1, H, FT), lambda i, j, be: (be[i], 0, j)),
            pl.BlockSpec((1, FT, H), lambda i, j, be: (be[i], j, 0)),
            pl.BlockSpec((1, BBLK, _LANES), lambda i, j, be: (i, 0, 0)),
        ],
        out_specs=pl.BlockSpec((BBLK, H), lambda i, j, be: (i, 0)),
    )
    return pl.pallas_call(
        _gmlp_body,
        grid_spec=grid_spec,
        out_shape=jax.ShapeDtypeStruct((NB, H), jnp.float32),
    )(be, xs, w_gate, w_up, w_down, ws3)


_TPW = T // 32
_CT = 16


def _make_combine():
    mesh = plsc.VectorSubcoreMesh(core_axis_name="c", subcore_axis_name="s")

    @functools.partial(
        pl.kernel,
        mesh=mesh,
        out_type=jax.ShapeDtypeStruct((T, H), jnp.float32),
        scratch_types=[
            pltpu.VMEM((2 * _CT,), jnp.int32),
            pltpu.VMEM((2 * _CT, H), jnp.float32),
            pltpu.VMEM((_CT, H), jnp.float32),
            pltpu.SemaphoreType.DMA,
        ],
    )
    def comb(ys_hbm, slots_hbm, out_hbm, slot_v, rows_v, out_v, sem):
        wid = lax.axis_index("s") * 2 + lax.axis_index("c")
        t0 = wid * _TPW

        def step(c, _):
            tb = t0 + c * _CT
            pltpu.sync_copy(slots_hbm.at[pl.ds(K * tb, K * _CT)], slot_v)
            pltpu.async_copy(ys_hbm.at[slot_v], rows_v, sem).wait()

            def row(r, _):
                for h in range(H // 16):
                    out_v[r, pl.ds(h * 16, 16)] = (
                        rows_v[2 * r, pl.ds(h * 16, 16)]
                        + rows_v[2 * r + 1, pl.ds(h * 16, 16)])
                return 0

            lax.fori_loop(0, _CT, row, 0)
            pltpu.sync_copy(out_v, out_hbm.at[pl.ds(tb, _CT)])
            return 0

        lax.fori_loop(0, _TPW // _CT, step, 0)

    return comb


_dispatch = functools.cache(_make_dispatch)
_combine = functools.cache(_make_combine)


def kernel(hidden_states, w_router, w_gate, w_up, w_down):
    shp = hidden_states.shape
    x = hidden_states.reshape(T, H)
    wr_pad = jnp.zeros((H, _LANES), jnp.float32).at[:, :E].set(w_router)

    logits128, slots128, wp128, be128 = _router_call(x, wr_pad)
    router_logits = logits128[:, :E]
    slots = slots128[:, :K].reshape(NP)
    w128 = jnp.broadcast_to(wp128[:, :K].reshape(NP, 1), (NP, _LANES))
    be = be128[:NBLK, 0]

    src = jnp.arange(NP, dtype=jnp.int32) // K
    xs, ws = _dispatch()(x, src, slots, w128)
    ys = _gmlp_call(be, xs, w_gate, w_up, w_down, ws.reshape(NBLK, BBLK, _LANES))
    out = _combine()(ys, slots)
    return out.reshape(shp), router_logits
